# Initial kernel scaffold; baseline (speedup 1.0000x reference)
import jax
import jax.numpy as jnp
from jax.experimental import pallas as pl


def kernel(seq_len, table):
    raise NotImplementedError("write your pallas kernel here")


# speedup vs baseline: 5.8677x; 5.8677x over previous
import functools

import jax
import jax.numpy as jnp
from jax import lax
from jax.experimental import pallas as pl
from jax.experimental.pallas import tpu as pltpu
from jax.experimental.pallas import tpu_sc as plsc

_MAXD = 128
_H = 12
_S = 2048
_T = 2 * _MAXD + 1
_TF = _T * _H
_ROWF = _S * _H
_FLATS = 4096 * _H
_TA = (_S - 1 - _MAXD) * _H
_PRE_END = 23040
_SUF_A = 26112
_NC = 2
_NS = 16
_NW = _NC * _NS
_PAIRS_PER_W = (_S // 2) // _NW
_FIRE = 4


@functools.partial(
    pl.kernel,
    out_type=jax.ShapeDtypeStruct((_S * _S * _H,), jnp.float32),
    mesh=plsc.VectorSubcoreMesh(
        core_axis_name="c", subcore_axis_name="s",
        num_cores=_NC, num_subcores=_NS,
    ),
    scratch_types=[
        pltpu.VMEM((3088,), jnp.float32),
        pltpu.VMEM((_FLATS,), jnp.float32),
        pltpu.VMEM((_FLATS,), jnp.float32),
        pltpu.SemaphoreType.DMA,
    ],
)
def _bias_kernel(table_hbm, out_hbm, tbl_v, sa, sb, sem):
    pltpu.async_copy(table_hbm, tbl_v.at[pl.ds(0, _TF)], sem).wait()

    t0pad = tbl_v[pl.ds(0, 16)]
    t256pad = tbl_v[pl.ds(_TF - 16, 16)]
    vpre, vsuf = [], []
    for k in range(3):
        idx = jnp.asarray(((jnp.arange(16) + 16 * k) % _H).astype(jnp.int32))
        vpre.append(t0pad.at[idx].get(mode="promise_in_bounds"))
        vsuf.append(t256pad.at[idx + 4].get(mode="promise_in_bounds"))

    def fill(j, carry):
        b = 48 * j
        for k in range(3):
            o = b + 16 * k
            sa[pl.ds(o, 16)] = vpre[k]
            sb[pl.ds(o, 16)] = vpre[k]
            sa[pl.ds(_SUF_A + o, 16)] = vsuf[k]
            sb[pl.ds(_SUF_A - 12 + o, 16)] = vsuf[k]
        return carry

    lax.fori_loop(0, 480, fill, 0)

    def body_a(j, carry):
        sa[pl.ds(_PRE_END + 16 * j, 16)] = tbl_v[pl.ds(_H + 16 * j, 16)]
        return carry

    lax.fori_loop(0, 192, body_a, 0)

    def body_b(j, carry):
        sb[pl.ds(_TA - _H + 16 * j, 16)] = tbl_v[pl.ds(16 * j, 16)]
        return carry

    lax.fori_loop(0, 192, body_b, 0)
    sb[pl.ds(_SUF_A - 28, 16)] = tbl_v[pl.ds(_TF - 16, 16)]

    wid = lax.axis_index("s") * _NC + lax.axis_index("c")
    p0 = wid * _PAIRS_PER_W

    def out_body(b, carry):
        copies = []
        for r in range(_FIRE):
            p = p0 + b * _FIRE + r
            m = (_S // 2 - 1) - p
            srcb = sb.at[pl.ds(m * 24, _ROWF)]
            srca = sa.at[pl.ds(m * 24, _ROWF)]
            copies.append(pltpu.async_copy(
                srcb, out_hbm.at[pl.ds(p * (2 * _ROWF), _ROWF)], sem))
            copies.append(pltpu.async_copy(
                srca, out_hbm.at[pl.ds(p * (2 * _ROWF) + _ROWF, _ROWF)], sem))
        for cp in copies:
            cp.wait()
        return carry

    lax.fori_loop(0, _PAIRS_PER_W // _FIRE, out_body, 0)


def kernel(seq_len, table):
    del seq_len
    out = _bias_kernel(table.reshape(_TF))
    return out.reshape(_S, _S, _H)


# speedup vs baseline: 5.8712x; 1.0006x over previous
import functools

import jax
import jax.numpy as jnp
from jax import lax
from jax.experimental import pallas as pl
from jax.experimental.pallas import tpu as pltpu
from jax.experimental.pallas import tpu_sc as plsc

_MAXD = 128
_H = 12
_S = 2048
_T = 2 * _MAXD + 1
_TF = _T * _H
_ROWF = _S * _H
_FLATS = 4096 * _H
_TA = (_S - 1 - _MAXD) * _H
_PRE_END = 23040
_SUF_A = 26112
_NC = 2
_NS = 16
_NW = _NC * _NS
_PAIRS_PER_W = (_S // 2) // _NW
_FIRE = 4


@functools.partial(
    pl.kernel,
    out_type=jax.ShapeDtypeStruct((_S * _S * _H,), jnp.float32),
    mesh=plsc.VectorSubcoreMesh(
        core_axis_name="c", subcore_axis_name="s",
        num_cores=_NC, num_subcores=_NS,
    ),
    scratch_types=[
        pltpu.VMEM((3088,), jnp.float32),
        pltpu.VMEM((_FLATS,), jnp.float32),
        pltpu.VMEM((_FLATS,), jnp.float32),
        pltpu.SemaphoreType.DMA,
    ],
)
def _bias_kernel(table_hbm, out_hbm, tbl_v, sa, sb, sem):
    pltpu.async_copy(table_hbm, tbl_v.at[pl.ds(0, _TF)], sem).wait()

    t0pad = tbl_v[pl.ds(0, 16)]
    t256pad = tbl_v[pl.ds(_TF - 16, 16)]
    vpre, vsuf = [], []
    for k in range(3):
        idx = jnp.asarray(((jnp.arange(16) + 16 * k) % _H).astype(jnp.int32))
        vpre.append(t0pad.at[idx].get(mode="promise_in_bounds"))
        vsuf.append(t256pad.at[idx + 4].get(mode="promise_in_bounds"))

    def fill(j, carry):
        b = 48 * j
        for k in range(3):
            o = b + 16 * k
            sa[pl.ds(o, 16)] = vpre[k]
            sb[pl.ds(o, 16)] = vpre[k]
            sa[pl.ds(_SUF_A + o, 16)] = vsuf[k]
            sb[pl.ds(_SUF_A - 12 + o, 16)] = vsuf[k]
        return carry

    lax.fori_loop(0, 480, fill, 0)

    def body_a(j, carry):
        sa[pl.ds(_PRE_END + 16 * j, 16)] = tbl_v[pl.ds(_H + 16 * j, 16)]
        return carry

    lax.fori_loop(0, 192, body_a, 0)

    def body_b(j, carry):
        sb[pl.ds(_TA - _H + 16 * j, 16)] = tbl_v[pl.ds(16 * j, 16)]
        return carry

    lax.fori_loop(0, 192, body_b, 0)
    sb[pl.ds(_SUF_A - 28, 16)] = tbl_v[pl.ds(_TF - 16, 16)]

    wid = lax.axis_index("s") * _NC + lax.axis_index("c")
    p0 = wid * _PAIRS_PER_W

    copies = []
    for r in range(_PAIRS_PER_W):
        p = p0 + r
        m = (_S // 2 - 1) - p
        srcb = sb.at[pl.ds(m * 24, _ROWF)]
        srca = sa.at[pl.ds(m * 24, _ROWF)]
        copies.append(pltpu.async_copy(
            srcb, out_hbm.at[pl.ds(p * (2 * _ROWF), _ROWF)], sem))
        copies.append(pltpu.async_copy(
            srca, out_hbm.at[pl.ds(p * (2 * _ROWF) + _ROWF, _ROWF)], sem))
    for cp in copies:
        cp.wait()


def kernel(seq_len, table):
    del seq_len
    out = _bias_kernel(table.reshape(_TF))
    return out.reshape(_S, _S, _H)


# speedup vs baseline: 38.5923x; 6.5732x over previous
import functools

import jax
import jax.numpy as jnp
from jax import lax
from jax.experimental import pallas as pl
from jax.experimental.pallas import tpu as pltpu
from jax.experimental.pallas import tpu_sc as plsc

_MAXD = 128
_H = 12
_S = 2048
_T = 2 * _MAXD + 1
_TTF = _H * _T
_NC, _NS = 2, 16
_NW = _NC * _NS
_NI = _S // 8
_PAIRS = _H * _NI
_PPW = _PAIRS // _NW


@functools.partial(
    pl.kernel,
    out_type=jax.ShapeDtypeStruct((_H, _NI, 16, 8, 128), jnp.float32),
    mesh=plsc.VectorSubcoreMesh(
        core_axis_name="c", subcore_axis_name="s",
        num_cores=_NC, num_subcores=_NS,
    ),
    scratch_types=[
        pltpu.VMEM((3104,), jnp.float32),
        pltpu.VMEM((2 * 4096,), jnp.float32),
        pltpu.VMEM((16, 8, 128), jnp.float32),
        pltpu.VMEM((16, 8, 128), jnp.float32),
        pltpu.SemaphoreType.DMA,
    ],
)
def _bias_kernel(tbl_hbm, out_hbm, tbl_v, st, stg0, stg1, sem):
    pltpu.async_copy(tbl_hbm, tbl_v.at[pl.ds(0, _TTF)], sem).wait()

    wid = lax.axis_index("s") * _NC + lax.axis_index("c")
    n0 = wid * _PPW
    h0 = (3 * wid) // 8

    for hh in range(2):
        h = jnp.minimum(h0 + hh, _H - 1)
        tb = h * _T
        base = hh * 4096
        pre = lax.broadcast_in_dim(tbl_v[pl.ds(tb, 16)][0], (16,), ())
        suf = lax.broadcast_in_dim(tbl_v[pl.ds(tb + _T - 16, 16)][15], (16,), ())

        def prefill(j, carry, base=base, pre=pre):
            st[pl.ds(base + 16 * j, 16)] = pre
            return carry

        lax.fori_loop(0, 120, prefill, 0)

        def suffill(j, carry, base=base, suf=suf):
            st[pl.ds(base + 2176 + 16 * j, 16)] = suf
            return carry

        lax.fori_loop(0, 120, suffill, 0)

        def body(j, carry, base=base, tb=tb):
            st[pl.ds(base + 1919 + 16 * j, 16)] = tbl_v[pl.ds(tb + 16 * j, 16)]
            return carry

        lax.fori_loop(0, 17, body, 0)

        def resuf(j, carry, base=base, suf=suf):
            st[pl.ds(base + 2176 + 16 * j, 16)] = suf
            return carry

        lax.fori_loop(0, 1, resuf, 0)

    def stage(q, stg):
        n = n0 + q
        h = n // _NI
        I = n - _NI * h
        hh = h - h0
        W0 = hh * 4096 + 2047 - 8 * I

        def sj(J, carry):
            def sr(r, carry2):
                w = W0 + 128 * J - r
                for u in range(8):
                    stg[J, r, pl.ds(16 * u, 16)] = st[pl.ds(w + 16 * u, 16)]
                return carry2

            lax.fori_loop(0, 8, sr, 0)
            return carry

        lax.fori_loop(0, 16, sj, 0)
        return out_hbm.at[h, I]

    d0 = pltpu.async_copy(stg0, stage(0, stg0), sem)
    dst1 = stage(1, stg1)
    d1 = pltpu.async_copy(stg1, dst1, sem)
    d0.wait()

    def out_body(q2, carry):
        q = 2 * q2
        dst = stage(q, stg0)
        da = pltpu.async_copy(stg0, dst, sem)
        d1.wait()
        dstb = stage(q + 1, stg1)
        db = pltpu.async_copy(stg1, dstb, sem)
        da.wait()
        return carry

    lax.fori_loop(1, _PPW // 2, out_body, 0)
    d1.wait()


def kernel(seq_len, table):
    del seq_len
    out = _bias_kernel(table.T.reshape(_TTF))
    return out.transpose(1, 3, 2, 4, 0).reshape(_S, _S, _H)


# speedup vs baseline: 168.5779x; 4.3682x over previous
import functools

import jax
import jax.numpy as jnp
from jax import lax
from jax.experimental import pallas as pl
from jax.experimental.pallas import tpu as pltpu
from jax.experimental.pallas import tpu_sc as plsc

_MAXD = 128
_H = 12
_S = 2048
_T = 2 * _MAXD + 1
_TTF = _H * _T
_NC, _NS = 2, 16
_NW = _NC * _NS
_NI = _S // 8
_PAIRS = _H * _NI
_PPW = _PAIRS // _NW


@functools.partial(
    pl.kernel,
    out_type=jax.ShapeDtypeStruct((_H, _NI, 16, 8, 128), jnp.float32),
    mesh=plsc.VectorSubcoreMesh(
        core_axis_name="c", subcore_axis_name="s",
        num_cores=_NC, num_subcores=_NS,
    ),
    scratch_types=[
        pltpu.VMEM((3104,), jnp.float32),
        pltpu.VMEM((2 * 4096,), jnp.float32),
        pltpu.VMEM((2, 16, 8, 128), jnp.float32),
        pltpu.SemaphoreType.DMA,
    ],
)
def _bias_kernel(tbl_hbm, out_hbm, tbl_v, st, stg, sem):
    pltpu.async_copy(tbl_hbm, tbl_v.at[pl.ds(0, _TTF)], sem).wait()

    wid = lax.axis_index("s") * _NC + lax.axis_index("c")
    n0 = wid * _PPW
    h0 = (3 * wid) // 8

    for hh in range(2):
        h = jnp.minimum(h0 + hh, _H - 1)
        tb = h * _T
        base = hh * 4096
        pre = lax.broadcast_in_dim(tbl_v[pl.ds(tb, 16)][0], (16,), ())
        suf = lax.broadcast_in_dim(tbl_v[pl.ds(tb + _T - 16, 16)][15], (16,), ())

        def prefill(j, carry, base=base, pre=pre):
            st[pl.ds(base + 16 * j, 16)] = pre
            return carry

        lax.fori_loop(0, 120, prefill, 0)

        def suffill(j, carry, base=base, suf=suf):
            st[pl.ds(base + 2176 + 16 * j, 16)] = suf
            return carry

        lax.fori_loop(0, 120, suffill, 0)

        def body(j, carry, base=base, tb=tb):
            st[pl.ds(base + 1919 + 16 * j, 16)] = tbl_v[pl.ds(tb + 16 * j, 16)]
            return carry

        lax.fori_loop(0, 17, body, 0)

        def resuf(j, carry, base=base, suf=suf):
            st[pl.ds(base + 2176 + 16 * j, 16)] = suf
            return carry

        lax.fori_loop(0, 1, resuf, 0)

    def out_body(q, carry):
        @pl.when(q >= 2)
        def _():
            pltpu.make_async_copy(out_hbm.at[0, 0], stg.at[0], sem).wait()

        n = n0 + q
        h = n // _NI
        I = n - _NI * h
        hh = h - h0
        W0 = hh * 4096 + 2047 - 8 * I
        b = lax.rem(q, 2)

        @plsc.parallel_loop(0, 16, step=1, unroll=2)
        def sj(J):
            w0 = W0 + 128 * J
            for r in range(8):
                for u in range(8):
                    stg[b, J, r, pl.ds(16 * u, 16)] = st[pl.ds(w0 - r + 16 * u, 16)]

        pltpu.async_copy(stg.at[b], out_hbm.at[h, I], sem)
        return carry

    lax.fori_loop(0, _PPW, out_body, 0)
    pltpu.make_async_copy(out_hbm.at[0, 0], stg.at[0], sem).wait()
    pltpu.make_async_copy(out_hbm.at[0, 0], stg.at[0], sem).wait()


def kernel(seq_len, table):
    del seq_len
    out = _bias_kernel(table.T.reshape(_TTF))
    return out.transpose(1, 3, 2, 4, 0).reshape(_S, _S, _H)
